# baseline (speedup 1.0000x reference)
import jax
import jax.numpy as jnp
from jax import lax
from jax.experimental import pallas as pl
from jax.experimental.pallas import tpu as pltpu
from jax.experimental.pallas import tpu_sc as plsc

NUM_USERS = 12500
NUM_ITEMS = 37500
N_NODES = 50000
NP = 50176
N_EDGES = 800000
N_EDGES_PAD = 819200
DIM = 64
HALF = 32
BATCH = 16384
L2 = 0.0001

NC = 2
NS = 16
CH = 128
EPT = N_EDGES_PAD // NS
NCHUNK = EPT // CH
BLK = 8
NBLK = NCHUNK // BLK
RPT = NP // NS
WCH = 112
NWCH = RPT // WCH
BPS = BATCH // NS


def _sc_body(emb0, rowi, coli, ev, user, pos, neg,
             e0s, e1, e2, e3, ppos, pneg, rparts,
             acc, rowB, colB, evB, grows, zb, rbuf,
             semg, sems_, semi):
  c = lax.axis_index("c")
  s = lax.axis_index("s")
  wid = c * NS + s
  cb = (grows.at[0, pl.ds(0, WCH)], grows.at[1, pl.ds(0, WCH)])

  def zinit(j, _):
    zb[j, pl.ds(0, 16)] = jnp.zeros((16,), jnp.float32)
    zb[j, pl.ds(16, 16)] = jnp.zeros((16,), jnp.float32)
    return _
  lax.fori_loop(0, WCH, zinit, 0)

  def rclamp(t):
    return jnp.minimum(s * RPT + t * WCH, N_NODES - WCH)

  li, so, zi = {}, {}, {}
  li[0] = pltpu.async_copy(
      emb0.at[pl.ds(rclamp(0), WCH), pl.ds(c * HALF, HALF)], cb[0], semg)
  for t in range(NWCH):
    r0 = s * RPT + t * WCH
    rc = rclamp(t)
    li[t].wait()
    if t >= 1:
      so[t - 1].wait()
    zi[t] = pltpu.async_copy(zb, acc.at[pl.ds(r0, WCH)], semi)
    if t >= 7:
      zi[t - 7].wait()
    if t < NWCH - 1:
      li[t + 1] = pltpu.async_copy(
          emb0.at[pl.ds(rclamp(t + 1), WCH), pl.ds(c * HALF, HALF)],
          cb[(t + 1) % 2], semg)
    so[t] = pltpu.async_copy(cb[t % 2], e0s.at[c, pl.ds(rc, WCH)], sems_)
  so[NWCH - 1].wait()
  for t in range(NWCH - 7, NWCH):
    zi[t].wait()
  plsc.subcore_barrier()

  def edge_pass(src):
    def scale_slot(gslot, islot, j):
      def sgroup(g, _):
        evv = evB[islot, j, pl.ds(g * 16, 16)]
        for i in range(16):
          e = g * 16 + i
          grows[gslot, e, pl.ds(0, 16)] = grows[gslot, e, pl.ds(0, 16)] * evv[i]
          grows[gslot, e, pl.ds(16, 16)] = grows[gslot, e, pl.ds(16, 16)] * evv[i]
        return _
      lax.fori_loop(0, CH // 16, sgroup, 0, unroll=2)

    t00 = s * NCHUNK
    pltpu.sync_copy(rowi.at[pl.ds(t00, BLK)], rowB.at[0])
    pltpu.sync_copy(coli.at[pl.ds(t00, BLK)], colB.at[0])
    pltpu.sync_copy(ev.at[pl.ds(t00, BLK)], evB.at[0])

    def block(b, _):
      islot = b % 2
      nslot = (b + 1) % 2
      tn = s * NCHUNK + jnp.minimum(b + 1, NBLK - 1) * BLK
      dr = pltpu.async_copy(rowi.at[pl.ds(tn, BLK)], rowB.at[nslot], semi)
      dc = pltpu.async_copy(coli.at[pl.ds(tn, BLK)], colB.at[nslot], semi)
      de = pltpu.async_copy(ev.at[pl.ds(tn, BLK)], evB.at[nslot], semi)
      d = {}
      sc = {}
      for j in range(4):
        d[j] = pltpu.async_copy(src.at[c].at[colB.at[islot, j]],
                                grows.at[j % 5], semg)
      for j in range(BLK):
        gslot = j % 5
        d[j].wait()
        if j >= 1:
          sc[j - 1].wait()
        if j + 4 < BLK:
          d[j + 4] = pltpu.async_copy(src.at[c].at[colB.at[islot, j + 4]],
                                      grows.at[(j + 4) % 5], semg)
        scale_slot(gslot, islot, j)
        sc[j] = pltpu.async_copy(grows.at[gslot], acc.at[rowB.at[islot, j]],
                                 sems_, add=True)
      sc[BLK - 1].wait()
      dr.wait()
      dc.wait()
      de.wait()
      return _
    lax.fori_loop(0, NBLK, block, 0)

  def write_out(dst):
    ci, co, zi = {}, {}, {}
    ci[0] = pltpu.async_copy(acc.at[pl.ds(s * RPT, WCH)], cb[0], semg)
    for t in range(NWCH):
      r0 = s * RPT + t * WCH
      ci[t].wait()
      if t >= 1:
        co[t - 1].wait()
      zi[t] = pltpu.async_copy(zb, acc.at[pl.ds(r0, WCH)], semi)
      if t >= 7:
        zi[t - 7].wait()
      if t < NWCH - 1:
        ci[t + 1] = pltpu.async_copy(
            acc.at[pl.ds(s * RPT + (t + 1) * WCH, WCH)], cb[(t + 1) % 2], semg)
      co[t] = pltpu.async_copy(cb[t % 2], dst.at[c, pl.ds(r0, WCH)], sems_)
    co[NWCH - 1].wait()
    for t in range(NWCH - 7, NWCH):
      zi[t].wait()

  for src, dst in ((e0s, e1), (e1, e2), (e2, e3)):
    edge_pass(src)
    plsc.subcore_barrier()
    write_out(dst)
    plsc.subcore_barrier()

  bufs = (e0s, e1, e2, e3)
  g0, g1, g2, g3, g4 = (grows.at[k] for k in range(5))
  uib = rowB.at[0, 0]
  pib = rowB.at[0, 1]
  nib = rowB.at[0, 2]
  prodp = grows.at[2, pl.ds(0, CH), pl.ds(0, 16)]
  prodn = grows.at[3, pl.ds(0, CH), pl.ds(0, 16)]

  def gath(buf, idx_ref, dst, sem):
    return pltpu.async_copy(buf.at[c].at[idx_ref], dst, sem)

  def addo(dst_ref, delta):
    def body(j, _):
      sl = pl.ds(j * 16, 16)
      dst_ref[sl] = dst_ref[sl] + delta
      return _
    lax.fori_loop(0, CH // 16, body, 0)

  def sum4(out, a, b, d):
    def body(j, _):
      for h in range(2):
        sl = pl.ds(h * 16, 16)
        out[j, sl] = (out[j, sl] + a[j, sl]) + (b[j, sl] + d[j, sl])
      return _
    lax.fori_loop(0, CH, body, 0, unroll=4)

  def prodl(out, x, y):
    def body(j, _):
      out[j, pl.ds(0, 16)] = (x[j, pl.ds(0, 16)] * y[j, pl.ds(0, 16)]
                              + x[j, pl.ds(16, 16)] * y[j, pl.ds(16, 16)])
      return _
    lax.fori_loop(0, CH, body, 0)

  def score_chunk(k, rv):
    off = s * BPS + k * CH
    du = pltpu.async_copy(user.at[pl.ds(off, CH)], uib, semi)
    dp = pltpu.async_copy(pos.at[pl.ds(off, CH)], pib, semi)
    dn = pltpu.async_copy(neg.at[pl.ds(off, CH)], nib, semi)
    du.wait(); dp.wait(); dn.wait()
    addo(pib, NUM_USERS)
    addo(nib, NUM_USERS)

    ds_ = [gath(bufs[b], uib, (g1, g2, g3, g4)[b], semg) for b in range(4)]
    for dd in ds_:
      dd.wait()
    sum4(g1, g2, g3, g4)
    ds_ = [gath(bufs[b], pib, (g0, g2, g3, g4)[b], semg) for b in range(4)]
    for dd in ds_:
      dd.wait()
    sum4(g0, g2, g3, g4)
    prodl(prodp, g1, g0)
    dpp = pltpu.async_copy(prodp, ppos.at[c, pl.ds(off, CH)], sems_)
    dpp.wait()
    ds_ = [gath(bufs[b], nib, (g0, g2, g3, g4)[b], semg) for b in range(4)]
    for dd in ds_:
      dd.wait()
    sum4(g0, g2, g3, g4)
    prodl(prodn, g1, g0)
    dpn = pltpu.async_copy(prodn, pneg.at[c, pl.ds(off, CH)], sems_)

    dpn.wait()
    dr = [gath(e0s, uib, g2, semg), gath(e0s, pib, g3, semg),
          gath(e0s, nib, g4, semg)]
    for dd in dr:
      dd.wait()

    def sq(j, a):
      for gg in (g2, g3, g4):
        v0 = gg[j, pl.ds(0, 16)]
        v1 = gg[j, pl.ds(16, 16)]
        a = a + v0 * v0 + v1 * v1
      return a
    rv = lax.fori_loop(0, CH, sq, rv)
    return rv

  rv = lax.fori_loop(0, BPS // CH, score_chunk, jnp.zeros((16,), jnp.float32))
  rbuf[...] = rv
  pltpu.sync_copy(rbuf, rparts.at[wid])


def _make_sc_kernel():
  mesh = plsc.VectorSubcoreMesh(core_axis_name="c", subcore_axis_name="s")
  f32 = jnp.float32
  out_type = (
      jax.ShapeDtypeStruct((NC, NP, HALF), f32),
      jax.ShapeDtypeStruct((NC, NP, HALF), f32),
      jax.ShapeDtypeStruct((NC, NP, HALF), f32),
      jax.ShapeDtypeStruct((NC, NP, HALF), f32),
      jax.ShapeDtypeStruct((NC, BATCH, 16), f32),
      jax.ShapeDtypeStruct((NC, BATCH, 16), f32),
      jax.ShapeDtypeStruct((NC * NS, 16), f32),
  )
  scratch = [
      pltpu.VMEM_SHARED((NP, HALF), f32),
      pltpu.VMEM((2, BLK, CH), jnp.int32),
      pltpu.VMEM((2, BLK, CH), jnp.int32),
      pltpu.VMEM((2, BLK, CH), f32),
      pltpu.VMEM((5, CH, HALF), f32),
      pltpu.VMEM((WCH, HALF), f32),
      pltpu.VMEM((16,), f32),
      pltpu.SemaphoreType.DMA,
      pltpu.SemaphoreType.DMA,
      pltpu.SemaphoreType.DMA,
  ]
  return pl.kernel(_sc_body, out_type=out_type, mesh=mesh,
                   scratch_types=scratch,
                   compiler_params=pltpu.CompilerParams(
                       use_tc_tiling_on_sc=False))


_sc_kernel = _make_sc_kernel()


def _tc_loss_body(pp_ref, pn_ref, rp_ref, loss_ref, reg_ref):
  pos = jnp.sum(pp_ref[0] + pp_ref[1], axis=-1)
  neg = jnp.sum(pn_ref[0] + pn_ref[1], axis=-1)
  d = (neg - pos) * (1.0 / 16.0)
  sp = jnp.maximum(d, 0.0) + jnp.log1p(jnp.exp(-jnp.abs(d)))
  loss_ref[...] = jnp.full((1, 1), jnp.mean(sp), jnp.float32)
  reg_ref[...] = jnp.full((1, 1), (0.5 * jnp.sum(rp_ref[...]) / float(BATCH)) * L2,
                          jnp.float32)


@jax.jit
def kernel(user_emb_w, item_emb_w, edge_values, edge_index, user, positive,
           negative):
  emb0 = jnp.concatenate([user_emb_w, item_emb_w], axis=0)
  pad = N_EDGES_PAD - N_EDGES
  rowi = jnp.concatenate([edge_index[0], jnp.zeros((pad,), jnp.int32)])
  coli = jnp.concatenate([edge_index[1], jnp.zeros((pad,), jnp.int32)])
  ev = jnp.concatenate([edge_values, jnp.zeros((pad,), jnp.float32)])
  rowi = rowi.reshape(N_EDGES_PAD // CH, CH)
  coli = coli.reshape(N_EDGES_PAD // CH, CH)
  ev = ev.reshape(N_EDGES_PAD // CH, CH)

  outs = _sc_kernel(emb0, rowi, coli, ev, user, positive, negative)
  ppos, pneg, rparts = outs[4], outs[5], outs[6]

  loss, reg = pl.pallas_call(
      _tc_loss_body,
      out_shape=(jax.ShapeDtypeStruct((1, 1), jnp.float32),
                 jax.ShapeDtypeStruct((1, 1), jnp.float32)),
  )(ppos, pneg, rparts)
  return (loss[0, 0], reg[0, 0])


# speedup vs baseline: 1.4106x; 1.4106x over previous
import jax
import jax.numpy as jnp
from jax import lax
from jax.experimental import pallas as pl
from jax.experimental.pallas import tpu as pltpu
from jax.experimental.pallas import tpu_sc as plsc

NUM_USERS = 12500
NUM_ITEMS = 37500
N_NODES = 50000
NP = 50176
N_EDGES = 800000
N_EDGES_PAD = 819200
DIM = 64
HALF = 32
BATCH = 16384
L2 = 0.0001

NC = 2
NS = 16
CH = 128
EPT = N_EDGES_PAD // NS
NCHUNK = EPT // CH
BLK = 8
NBLK = NCHUNK // BLK
RPT = NP // NS
WCH = 112
NWCH = RPT // WCH
BPS = BATCH // NS


def _sc_body(emb0, rowi, coli, ev, user, pos, neg,
             e0s, e1, e2, e3, ppos, pneg, rparts,
             acc, rowB, colB, evB, grows, zb, rbuf,
             semg, sems_, semi):
  c = lax.axis_index("c")
  s = lax.axis_index("s")
  wid = c * NS + s
  cb = (grows.at[0, pl.ds(0, WCH)], grows.at[1, pl.ds(0, WCH)])

  def zinit(j, _):
    zb[j, pl.ds(0, 16)] = jnp.zeros((16,), jnp.float32)
    zb[j, pl.ds(16, 16)] = jnp.zeros((16,), jnp.float32)
    return _
  lax.fori_loop(0, WCH, zinit, 0)

  def rclamp(t):
    return jnp.minimum(s * RPT + t * WCH, N_NODES - WCH)

  li, so, zi = {}, {}, {}
  li[0] = pltpu.async_copy(
      emb0.at[pl.ds(rclamp(0), WCH), pl.ds(c * HALF, HALF)], cb[0], semg)
  for t in range(NWCH):
    r0 = s * RPT + t * WCH
    rc = rclamp(t)
    li[t].wait()
    if t >= 1:
      so[t - 1].wait()
    zi[t] = pltpu.async_copy(zb, acc.at[pl.ds(r0, WCH)], semi)
    if t >= 7:
      zi[t - 7].wait()
    if t < NWCH - 1:
      li[t + 1] = pltpu.async_copy(
          emb0.at[pl.ds(rclamp(t + 1), WCH), pl.ds(c * HALF, HALF)],
          cb[(t + 1) % 2], semg)
    so[t] = pltpu.async_copy(cb[t % 2], e0s.at[c, pl.ds(rc, WCH)], sems_)
  so[NWCH - 1].wait()
  for t in range(NWCH - 7, NWCH):
    zi[t].wait()
  plsc.subcore_barrier()

  def edge_pass(src):
    def scale_slot(gslot, islot, j):
      def sgroup(g, _):
        evv = evB[islot, j, pl.ds(g * 16, 16)]
        for i in range(16):
          e = g * 16 + i
          grows[gslot, e, pl.ds(0, 16)] = grows[gslot, e, pl.ds(0, 16)] * evv[i]
          grows[gslot, e, pl.ds(16, 16)] = grows[gslot, e, pl.ds(16, 16)] * evv[i]
        return _
      lax.fori_loop(0, CH // 16, sgroup, 0)

    t00 = s * NCHUNK
    pltpu.sync_copy(rowi.at[pl.ds(t00, BLK)], rowB.at[0])
    pltpu.sync_copy(coli.at[pl.ds(t00, BLK)], colB.at[0])
    pltpu.sync_copy(ev.at[pl.ds(t00, BLK)], evB.at[0])

    def block(b, _):
      islot = b % 2
      nslot = (b + 1) % 2
      tn = s * NCHUNK + jnp.minimum(b + 1, NBLK - 1) * BLK
      dr = pltpu.async_copy(rowi.at[pl.ds(tn, BLK)], rowB.at[nslot], semi)
      dc = pltpu.async_copy(coli.at[pl.ds(tn, BLK)], colB.at[nslot], semi)
      de = pltpu.async_copy(ev.at[pl.ds(tn, BLK)], evB.at[nslot], semi)
      d = {}
      sc = {}
      for j in range(4):
        d[j] = pltpu.async_copy(src.at[c].at[colB.at[islot, j]],
                                grows.at[j % 5], semg)
      for j in range(BLK):
        gslot = j % 5
        d[j].wait()
        if j >= 1:
          sc[j - 1].wait()
        if j + 4 < BLK:
          d[j + 4] = pltpu.async_copy(src.at[c].at[colB.at[islot, j + 4]],
                                      grows.at[(j + 4) % 5], semg)
        scale_slot(gslot, islot, j)
        sc[j] = pltpu.async_copy(grows.at[gslot], acc.at[rowB.at[islot, j]],
                                 sems_, add=True)
      sc[BLK - 1].wait()
      dr.wait()
      dc.wait()
      de.wait()
      return _
    lax.fori_loop(0, NBLK, block, 0)

  def write_out(dst):
    ci, co, zi = {}, {}, {}
    ci[0] = pltpu.async_copy(acc.at[pl.ds(s * RPT, WCH)], cb[0], semg)
    for t in range(NWCH):
      r0 = s * RPT + t * WCH
      ci[t].wait()
      if t >= 1:
        co[t - 1].wait()
      zi[t] = pltpu.async_copy(zb, acc.at[pl.ds(r0, WCH)], semi)
      if t >= 7:
        zi[t - 7].wait()
      if t < NWCH - 1:
        ci[t + 1] = pltpu.async_copy(
            acc.at[pl.ds(s * RPT + (t + 1) * WCH, WCH)], cb[(t + 1) % 2], semg)
      co[t] = pltpu.async_copy(cb[t % 2], dst.at[c, pl.ds(r0, WCH)], sems_)
    co[NWCH - 1].wait()
    for t in range(NWCH - 7, NWCH):
      zi[t].wait()

  for src, dst in ((e0s, e1), (e1, e2), (e2, e3)):
    edge_pass(src)
    plsc.subcore_barrier()
    write_out(dst)
    plsc.subcore_barrier()

  bufs = (e0s, e1, e2, e3)
  g0, g1, g2, g3, g4 = (grows.at[k] for k in range(5))
  uib = rowB.at[0, 0]
  pib = rowB.at[0, 1]
  nib = rowB.at[0, 2]
  prodp = grows.at[2, pl.ds(0, CH), pl.ds(0, 16)]
  prodn = grows.at[3, pl.ds(0, CH), pl.ds(0, 16)]

  def gath(buf, idx_ref, dst, sem):
    return pltpu.async_copy(buf.at[c].at[idx_ref], dst, sem)

  def addo(dst_ref, delta):
    def body(j, _):
      sl = pl.ds(j * 16, 16)
      dst_ref[sl] = dst_ref[sl] + delta
      return _
    lax.fori_loop(0, CH // 16, body, 0)

  def sum4(out, a, b, d):
    def body(j, _):
      for h in range(2):
        sl = pl.ds(h * 16, 16)
        out[j, sl] = (out[j, sl] + a[j, sl]) + (b[j, sl] + d[j, sl])
      return _
    lax.fori_loop(0, CH, body, 0)

  def prodl(out, x, y):
    def body(j, _):
      out[j, pl.ds(0, 16)] = (x[j, pl.ds(0, 16)] * y[j, pl.ds(0, 16)]
                              + x[j, pl.ds(16, 16)] * y[j, pl.ds(16, 16)])
      return _
    lax.fori_loop(0, CH, body, 0)

  def score_chunk(k, rv):
    off = s * BPS + k * CH
    du = pltpu.async_copy(user.at[pl.ds(off, CH)], uib, semi)
    dp = pltpu.async_copy(pos.at[pl.ds(off, CH)], pib, semi)
    dn = pltpu.async_copy(neg.at[pl.ds(off, CH)], nib, semi)
    du.wait(); dp.wait(); dn.wait()
    addo(pib, NUM_USERS)
    addo(nib, NUM_USERS)

    ds_ = [gath(bufs[b], uib, (g1, g2, g3, g4)[b], semg) for b in range(4)]
    for dd in ds_:
      dd.wait()
    sum4(g1, g2, g3, g4)
    ds_ = [gath(bufs[b], pib, (g0, g2, g3, g4)[b], semg) for b in range(4)]
    for dd in ds_:
      dd.wait()
    sum4(g0, g2, g3, g4)
    prodl(prodp, g1, g0)
    dpp = pltpu.async_copy(prodp, ppos.at[c, pl.ds(off, CH)], sems_)
    dpp.wait()
    ds_ = [gath(bufs[b], nib, (g0, g2, g3, g4)[b], semg) for b in range(4)]
    for dd in ds_:
      dd.wait()
    sum4(g0, g2, g3, g4)
    prodl(prodn, g1, g0)
    dpn = pltpu.async_copy(prodn, pneg.at[c, pl.ds(off, CH)], sems_)

    dpn.wait()
    dr = [gath(e0s, uib, g2, semg), gath(e0s, pib, g3, semg),
          gath(e0s, nib, g4, semg)]
    for dd in dr:
      dd.wait()

    def sq(j, a):
      for gg in (g2, g3, g4):
        v0 = gg[j, pl.ds(0, 16)]
        v1 = gg[j, pl.ds(16, 16)]
        a = a + v0 * v0 + v1 * v1
      return a
    rv = lax.fori_loop(0, CH, sq, rv)
    return rv

  rv = lax.fori_loop(0, BPS // CH, score_chunk, jnp.zeros((16,), jnp.float32))
  rbuf[...] = rv
  pltpu.sync_copy(rbuf, rparts.at[wid])


def _make_sc_kernel():
  mesh = plsc.VectorSubcoreMesh(core_axis_name="c", subcore_axis_name="s")
  f32 = jnp.float32
  out_type = (
      jax.ShapeDtypeStruct((NC, NP, HALF), f32),
      jax.ShapeDtypeStruct((NC, NP, HALF), f32),
      jax.ShapeDtypeStruct((NC, NP, HALF), f32),
      jax.ShapeDtypeStruct((NC, NP, HALF), f32),
      jax.ShapeDtypeStruct((NC, BATCH, 16), f32),
      jax.ShapeDtypeStruct((NC, BATCH, 16), f32),
      jax.ShapeDtypeStruct((NC * NS, 16), f32),
  )
  scratch = [
      pltpu.VMEM_SHARED((NP, HALF), f32),
      pltpu.VMEM((2, BLK, CH), jnp.int32),
      pltpu.VMEM((2, BLK, CH), jnp.int32),
      pltpu.VMEM((2, BLK, CH), f32),
      pltpu.VMEM((5, CH, HALF), f32),
      pltpu.VMEM((WCH, HALF), f32),
      pltpu.VMEM((16,), f32),
      pltpu.SemaphoreType.DMA,
      pltpu.SemaphoreType.DMA,
      pltpu.SemaphoreType.DMA,
  ]
  return pl.kernel(_sc_body, out_type=out_type, mesh=mesh,
                   scratch_types=scratch,
                   compiler_params=pltpu.CompilerParams(
                       use_tc_tiling_on_sc=False))


_sc_kernel = _make_sc_kernel()


def _tc_loss_body(pp_ref, pn_ref, rp_ref, loss_ref, reg_ref):
  pos = jnp.sum(pp_ref[0] + pp_ref[1], axis=-1)
  neg = jnp.sum(pn_ref[0] + pn_ref[1], axis=-1)
  d = (neg - pos) * (1.0 / 16.0)
  sp = jnp.maximum(d, 0.0) + jnp.log1p(jnp.exp(-jnp.abs(d)))
  loss_ref[...] = jnp.full((1, 1), jnp.mean(sp), jnp.float32)
  reg_ref[...] = jnp.full((1, 1), (0.5 * jnp.sum(rp_ref[...]) / float(BATCH)) * L2,
                          jnp.float32)


@jax.jit
def kernel(user_emb_w, item_emb_w, edge_values, edge_index, user, positive,
           negative):
  emb0 = jnp.concatenate([user_emb_w, item_emb_w], axis=0)
  pad = N_EDGES_PAD - N_EDGES
  rowi = jnp.concatenate([edge_index[0], jnp.zeros((pad,), jnp.int32)])
  coli = jnp.concatenate([edge_index[1], jnp.zeros((pad,), jnp.int32)])
  ev = jnp.concatenate([edge_values, jnp.zeros((pad,), jnp.float32)])
  rowi = rowi.reshape(N_EDGES_PAD // CH, CH)
  coli = coli.reshape(N_EDGES_PAD // CH, CH)
  ev = ev.reshape(N_EDGES_PAD // CH, CH)

  outs = _sc_kernel(emb0, rowi, coli, ev, user, positive, negative)
  ppos, pneg, rparts = outs[4], outs[5], outs[6]

  loss, reg = pl.pallas_call(
      _tc_loss_body,
      out_shape=(jax.ShapeDtypeStruct((1, 1), jnp.float32),
                 jax.ShapeDtypeStruct((1, 1), jnp.float32)),
  )(ppos, pneg, rparts)
  return (loss[0, 0], reg[0, 0])


# speedup vs baseline: 1.4663x; 1.0395x over previous
import jax
import jax.numpy as jnp
from jax import lax
from jax.experimental import pallas as pl
from jax.experimental.pallas import tpu as pltpu
from jax.experimental.pallas import tpu_sc as plsc

NUM_USERS = 12500
NUM_ITEMS = 37500
N_NODES = 50000
NP = 50176
N_EDGES = 800000
N_EDGES_PAD = 819200
DIM = 64
HALF = 32
BATCH = 16384
L2 = 0.0001

NC = 2
NS = 16
CH = 128
EPT = N_EDGES_PAD // NS
NCHUNK = EPT // CH
BLK = 8
NBLK = NCHUNK // BLK
RPT = NP // NS
WCH = 112
NWCH = RPT // WCH
BPS = BATCH // NS


def _sc_body(emb0, rowi, coli, ev, user, pos, neg,
             e0s, e1, e2, e3, ppos, pneg, rparts,
             acc, rowB, colB, evB, grows, zb, rbuf,
             semg, sems_, semi):
  c = lax.axis_index("c")
  s = lax.axis_index("s")
  wid = c * NS + s
  cb = (grows.at[0, pl.ds(0, WCH)], grows.at[1, pl.ds(0, WCH)])

  def zinit(j, _):
    zb[j, pl.ds(0, 16)] = jnp.zeros((16,), jnp.float32)
    zb[j, pl.ds(16, 16)] = jnp.zeros((16,), jnp.float32)
    return _
  lax.fori_loop(0, WCH, zinit, 0)

  def rclamp(t):
    return jnp.minimum(s * RPT + t * WCH, N_NODES - WCH)

  li, so, zi = {}, {}, {}
  li[0] = pltpu.async_copy(
      emb0.at[pl.ds(rclamp(0), WCH), pl.ds(c * HALF, HALF)], cb[0], semg)
  for t in range(NWCH):
    r0 = s * RPT + t * WCH
    rc = rclamp(t)
    li[t].wait()
    if t >= 1:
      so[t - 1].wait()
    zi[t] = pltpu.async_copy(zb, acc.at[pl.ds(r0, WCH)], semi)
    if t >= 7:
      zi[t - 7].wait()
    if t < NWCH - 1:
      li[t + 1] = pltpu.async_copy(
          emb0.at[pl.ds(rclamp(t + 1), WCH), pl.ds(c * HALF, HALF)],
          cb[(t + 1) % 2], semg)
    so[t] = pltpu.async_copy(cb[t % 2], e0s.at[c, pl.ds(rc, WCH)], sems_)
  so[NWCH - 1].wait()
  for t in range(NWCH - 7, NWCH):
    zi[t].wait()
  plsc.subcore_barrier()

  def edge_pass(src):
    def scale_slot(gslot, islot, j):
      def sgroup(g, _):
        evv = evB[islot, j, pl.ds(g * 16, 16)]
        for i in range(16):
          e = g * 16 + i
          grows[gslot, e, pl.ds(0, 16)] = grows[gslot, e, pl.ds(0, 16)] * evv[i]
          grows[gslot, e, pl.ds(16, 16)] = grows[gslot, e, pl.ds(16, 16)] * evv[i]
        return _
      lax.fori_loop(0, CH // 16, sgroup, 0)

    t00 = s * NCHUNK
    pltpu.sync_copy(rowi.at[pl.ds(t00, BLK)], rowB.at[0])
    pltpu.sync_copy(coli.at[pl.ds(t00, BLK)], colB.at[0])
    pltpu.sync_copy(ev.at[pl.ds(t00, BLK)], evB.at[0])

    def block(b, _):
      islot = b % 2
      nslot = (b + 1) % 2
      tn = s * NCHUNK + jnp.minimum(b + 1, NBLK - 1) * BLK
      dr = pltpu.async_copy(rowi.at[pl.ds(tn, BLK)], rowB.at[nslot], semi)
      dc = pltpu.async_copy(coli.at[pl.ds(tn, BLK)], colB.at[nslot], semi)
      de = pltpu.async_copy(ev.at[pl.ds(tn, BLK)], evB.at[nslot], semi)
      d = {}
      sc = {}
      for j in range(3):
        d[j] = pltpu.async_copy(src.at[c].at[colB.at[islot, j]],
                                grows.at[j % 5], semg)
      for j in range(BLK):
        gslot = j % 5
        d[j].wait()
        if j >= 2:
          sc[j - 2].wait()
        if j + 3 < BLK:
          d[j + 3] = pltpu.async_copy(src.at[c].at[colB.at[islot, j + 3]],
                                      grows.at[(j + 3) % 5], semg)
        scale_slot(gslot, islot, j)
        sc[j] = pltpu.async_copy(grows.at[gslot], acc.at[rowB.at[islot, j]],
                                 sems_, add=True)
      sc[BLK - 2].wait()
      sc[BLK - 1].wait()
      dr.wait()
      dc.wait()
      de.wait()
      return _
    lax.fori_loop(0, NBLK, block, 0)

  def write_out(dst):
    ci, co, zi = {}, {}, {}
    ci[0] = pltpu.async_copy(acc.at[pl.ds(s * RPT, WCH)], cb[0], semg)
    for t in range(NWCH):
      r0 = s * RPT + t * WCH
      ci[t].wait()
      if t >= 1:
        co[t - 1].wait()
      zi[t] = pltpu.async_copy(zb, acc.at[pl.ds(r0, WCH)], semi)
      if t >= 7:
        zi[t - 7].wait()
      if t < NWCH - 1:
        ci[t + 1] = pltpu.async_copy(
            acc.at[pl.ds(s * RPT + (t + 1) * WCH, WCH)], cb[(t + 1) % 2], semg)
      co[t] = pltpu.async_copy(cb[t % 2], dst.at[c, pl.ds(r0, WCH)], sems_)
    co[NWCH - 1].wait()
    for t in range(NWCH - 7, NWCH):
      zi[t].wait()

  for src, dst in ((e0s, e1), (e1, e2), (e2, e3)):
    edge_pass(src)
    plsc.subcore_barrier()
    write_out(dst)
    plsc.subcore_barrier()

  bufs = (e0s, e1, e2, e3)
  g0, g1, g2, g3, g4 = (grows.at[k] for k in range(5))
  uib = rowB.at[0, 0]
  pib = rowB.at[0, 1]
  nib = rowB.at[0, 2]
  prodp = grows.at[2, pl.ds(0, CH), pl.ds(0, 16)]
  prodn = grows.at[3, pl.ds(0, CH), pl.ds(0, 16)]

  def gath(buf, idx_ref, dst, sem):
    return pltpu.async_copy(buf.at[c].at[idx_ref], dst, sem)

  def addo(dst_ref, delta):
    def body(j, _):
      sl = pl.ds(j * 16, 16)
      dst_ref[sl] = dst_ref[sl] + delta
      return _
    lax.fori_loop(0, CH // 16, body, 0)

  def sum4(out, a, b, d):
    def body(j, _):
      for h in range(2):
        sl = pl.ds(h * 16, 16)
        out[j, sl] = (out[j, sl] + a[j, sl]) + (b[j, sl] + d[j, sl])
      return _
    lax.fori_loop(0, CH, body, 0)

  def prodl(out, x, y):
    def body(j, _):
      out[j, pl.ds(0, 16)] = (x[j, pl.ds(0, 16)] * y[j, pl.ds(0, 16)]
                              + x[j, pl.ds(16, 16)] * y[j, pl.ds(16, 16)])
      return _
    lax.fori_loop(0, CH, body, 0)

  def score_chunk(k, rv):
    off = s * BPS + k * CH
    du = pltpu.async_copy(user.at[pl.ds(off, CH)], uib, semi)
    dp = pltpu.async_copy(pos.at[pl.ds(off, CH)], pib, semi)
    dn = pltpu.async_copy(neg.at[pl.ds(off, CH)], nib, semi)
    du.wait(); dp.wait(); dn.wait()
    addo(pib, NUM_USERS)
    addo(nib, NUM_USERS)

    ds_ = [gath(bufs[b], uib, (g1, g2, g3, g4)[b], semg) for b in range(4)]
    for dd in ds_:
      dd.wait()
    sum4(g1, g2, g3, g4)
    ds_ = [gath(bufs[b], pib, (g0, g2, g3, g4)[b], semg) for b in range(4)]
    for dd in ds_:
      dd.wait()
    sum4(g0, g2, g3, g4)
    prodl(prodp, g1, g0)
    dpp = pltpu.async_copy(prodp, ppos.at[c, pl.ds(off, CH)], sems_)
    dpp.wait()
    ds_ = [gath(bufs[b], nib, (g0, g2, g3, g4)[b], semg) for b in range(4)]
    for dd in ds_:
      dd.wait()
    sum4(g0, g2, g3, g4)
    prodl(prodn, g1, g0)
    dpn = pltpu.async_copy(prodn, pneg.at[c, pl.ds(off, CH)], sems_)

    dpn.wait()
    dr = [gath(e0s, uib, g2, semg), gath(e0s, pib, g3, semg),
          gath(e0s, nib, g4, semg)]
    for dd in dr:
      dd.wait()

    def sq(j, a):
      for gg in (g2, g3, g4):
        v0 = gg[j, pl.ds(0, 16)]
        v1 = gg[j, pl.ds(16, 16)]
        a = a + v0 * v0 + v1 * v1
      return a
    rv = lax.fori_loop(0, CH, sq, rv)
    return rv

  rv = lax.fori_loop(0, BPS // CH, score_chunk, jnp.zeros((16,), jnp.float32))
  rbuf[...] = rv
  pltpu.sync_copy(rbuf, rparts.at[wid])


def _make_sc_kernel():
  mesh = plsc.VectorSubcoreMesh(core_axis_name="c", subcore_axis_name="s")
  f32 = jnp.float32
  out_type = (
      jax.ShapeDtypeStruct((NC, NP, HALF), f32),
      jax.ShapeDtypeStruct((NC, NP, HALF), f32),
      jax.ShapeDtypeStruct((NC, NP, HALF), f32),
      jax.ShapeDtypeStruct((NC, NP, HALF), f32),
      jax.ShapeDtypeStruct((NC, BATCH, 16), f32),
      jax.ShapeDtypeStruct((NC, BATCH, 16), f32),
      jax.ShapeDtypeStruct((NC * NS, 16), f32),
  )
  scratch = [
      pltpu.VMEM_SHARED((NP, HALF), f32),
      pltpu.VMEM((2, BLK, CH), jnp.int32),
      pltpu.VMEM((2, BLK, CH), jnp.int32),
      pltpu.VMEM((2, BLK, CH), f32),
      pltpu.VMEM((5, CH, HALF), f32),
      pltpu.VMEM((WCH, HALF), f32),
      pltpu.VMEM((16,), f32),
      pltpu.SemaphoreType.DMA,
      pltpu.SemaphoreType.DMA,
      pltpu.SemaphoreType.DMA,
  ]
  return pl.kernel(_sc_body, out_type=out_type, mesh=mesh,
                   scratch_types=scratch,
                   compiler_params=pltpu.CompilerParams(
                       use_tc_tiling_on_sc=False))


_sc_kernel = _make_sc_kernel()


def _tc_loss_body(pp_ref, pn_ref, rp_ref, loss_ref, reg_ref):
  pos = jnp.sum(pp_ref[0] + pp_ref[1], axis=-1)
  neg = jnp.sum(pn_ref[0] + pn_ref[1], axis=-1)
  d = (neg - pos) * (1.0 / 16.0)
  sp = jnp.maximum(d, 0.0) + jnp.log1p(jnp.exp(-jnp.abs(d)))
  loss_ref[...] = jnp.full((1, 1), jnp.mean(sp), jnp.float32)
  reg_ref[...] = jnp.full((1, 1), (0.5 * jnp.sum(rp_ref[...]) / float(BATCH)) * L2,
                          jnp.float32)


@jax.jit
def kernel(user_emb_w, item_emb_w, edge_values, edge_index, user, positive,
           negative):
  emb0 = jnp.concatenate([user_emb_w, item_emb_w], axis=0)
  pad = N_EDGES_PAD - N_EDGES
  rowi = jnp.concatenate([edge_index[0], jnp.zeros((pad,), jnp.int32)])
  coli = jnp.concatenate([edge_index[1], jnp.zeros((pad,), jnp.int32)])
  ev = jnp.concatenate([edge_values, jnp.zeros((pad,), jnp.float32)])
  rowi = rowi.reshape(N_EDGES_PAD // CH, CH)
  coli = coli.reshape(N_EDGES_PAD // CH, CH)
  ev = ev.reshape(N_EDGES_PAD // CH, CH)

  outs = _sc_kernel(emb0, rowi, coli, ev, user, positive, negative)
  ppos, pneg, rparts = outs[4], outs[5], outs[6]

  loss, reg = pl.pallas_call(
      _tc_loss_body,
      out_shape=(jax.ShapeDtypeStruct((1, 1), jnp.float32),
                 jax.ShapeDtypeStruct((1, 1), jnp.float32)),
  )(ppos, pneg, rparts)
  return (loss[0, 0], reg[0, 0])
